# Initial kernel scaffold; baseline (speedup 1.0000x reference)
import jax
import jax.numpy as jnp
from jax.experimental import pallas as pl


def kernel(z, W_in, b_in, W_out, b_out, v0, v1, v2, v3, v4):
    raise NotImplementedError("write your pallas kernel here")


# speedup vs baseline: 4.6438x; 4.6438x over previous
import jax
import jax.numpy as jnp
from jax.experimental import pallas as pl
from jax.experimental.pallas import tpu as pltpu

_LEVELS = (8, 8, 8, 6, 5)
_CD = 5
_LANES = 128
_MAXLEV = 8
_BM = 512


def _fused(z_ref, win_ref, bin_ref, lo_ref, inv_ref, maxi_ref, vtab_ref,
           coefa_ref, coefb_ref, mask_ref, wout_ref, bout_ref,
           out_ref, idx_ref, loss_ref):
    zp = jnp.dot(z_ref[...], win_ref[...],
                 preferred_element_type=jnp.float32) + bin_ref[...]
    t = (zp - lo_ref[...]) * inv_ref[...]
    k = jnp.clip(jnp.round(t), 0.0, maxi_ref[...])
    q = jnp.zeros_like(zp)
    for kk in range(_MAXLEV):
        q = jnp.where(k == float(kk), vtab_ref[kk, :][None, :], q)
    err = (zp - q) * mask_ref[...]
    blk = jnp.sum(err * err)

    @pl.when(pl.program_id(0) == 0)
    def _():
        loss_ref[...] = jnp.zeros((1, 1), jnp.float32)

    loss_ref[...] += blk.reshape(1, 1)
    idx_ref[...] = jnp.sum(q * coefa_ref[...] + coefb_ref[...],
                           axis=1, keepdims=True)
    out_ref[...] = jnp.dot(q, wout_ref[...],
                           preferred_element_type=jnp.float32) + bout_ref[...]


def kernel(z, W_in, b_in, W_out, b_out, v0, v1, v2, v3, v4):
    values = [v0, v1, v2, v3, v4]
    b, n, dim = z.shape
    m = b * n
    cd = _CD

    win_p = jnp.zeros((dim, _LANES), jnp.float32).at[:, :cd].set(W_in.T)
    wout_p = jnp.zeros((_LANES, dim), jnp.float32).at[:cd, :].set(W_out.T)
    bin_p = jnp.zeros((1, _LANES), jnp.float32).at[0, :cd].set(b_in)
    bout_p = b_out.reshape(1, dim)

    vtab = jnp.zeros((_MAXLEV, _LANES), jnp.float32)
    lo = jnp.zeros((1, _LANES), jnp.float32)
    inv = jnp.zeros((1, _LANES), jnp.float32)
    maxi = jnp.zeros((1, _LANES), jnp.float32)
    for i, lv in enumerate(_LEVELS):
        vtab = vtab.at[:lv, i].set(values[i])
        lo = lo.at[0, i].set(values[i][0])
        step = values[i][1] - values[i][0]
        inv = inv.at[0, i].set(1.0 / step)
        maxi = maxi.at[0, i].set(float(lv - 1))

    levels = jnp.array(_LEVELS, jnp.int32)
    basis = jnp.concatenate(
        [jnp.array([1], jnp.int32), jnp.cumprod(levels[:-1])])
    half = (levels // 2).astype(jnp.float32)
    basis_f = basis.astype(jnp.float32)
    coefa = jnp.zeros((1, _LANES), jnp.float32).at[0, :cd].set(
        2.0 * half * basis_f)
    coefb = jnp.zeros((1, _LANES), jnp.float32).at[0, :cd].set(
        half * basis_f)
    mask = jnp.zeros((1, _LANES), jnp.float32).at[0, :cd].set(1.0)

    zf = z.reshape(m, dim)
    grid = (m // _BM,)
    full = lambda i: (0, 0)
    out, idx, loss = pl.pallas_call(
        _fused,
        grid=grid,
        in_specs=[
            pl.BlockSpec((_BM, dim), lambda i: (i, 0)),
            pl.BlockSpec((dim, _LANES), full),
            pl.BlockSpec((1, _LANES), full),
            pl.BlockSpec((1, _LANES), full),
            pl.BlockSpec((1, _LANES), full),
            pl.BlockSpec((1, _LANES), full),
            pl.BlockSpec((_MAXLEV, _LANES), full),
            pl.BlockSpec((1, _LANES), full),
            pl.BlockSpec((1, _LANES), full),
            pl.BlockSpec((1, _LANES), full),
            pl.BlockSpec((_LANES, dim), full),
            pl.BlockSpec((1, dim), full),
        ],
        out_specs=[
            pl.BlockSpec((_BM, dim), lambda i: (i, 0)),
            pl.BlockSpec((_BM, 1), lambda i: (i, 0)),
            pl.BlockSpec((1, 1), full),
        ],
        out_shape=[
            jax.ShapeDtypeStruct((m, dim), jnp.float32),
            jax.ShapeDtypeStruct((m, 1), jnp.float32),
            jax.ShapeDtypeStruct((1, 1), jnp.float32),
        ],
        compiler_params=pltpu.CompilerParams(
            dimension_semantics=("arbitrary",)),
    )(zf, win_p, bin_p, lo, inv, maxi, vtab, coefa, coefb, mask,
      wout_p, bout_p)

    out = out.reshape(b, n, dim)
    indices = idx.reshape(b, n)
    loss_val = loss[0, 0] * (0.2 / (m * cd))
    return out, indices, loss_val
